# baseline (speedup 1.0000x reference)
import functools
import math

import jax
import jax.numpy as jnp
from jax import lax
from jax.experimental import pallas as pl
from jax.experimental.pallas import tpu as pltpu
from jax.experimental.pallas import tpu_sc as plsc

_C = math.sqrt(2.0 / math.pi)
_S1 = -2.0 * _C
_S2 = -2.0 * _C * 0.044715

_D = 2048
_NW = 32
_ROWS = 8192
_RPW = _ROWS // _NW
_CHW = 8
_SUB = 2


@functools.partial(
    pl.kernel,
    mesh=plsc.VectorSubcoreMesh(core_axis_name="c", subcore_axis_name="s"),
    out_type=jax.ShapeDtypeStruct((_ROWS, _D), jnp.float32),
    scratch_types=[
        pltpu.VMEM((_CHW, _D), jnp.float32),
        pltpu.VMEM((_CHW, _D), jnp.float32),
    ],
)
def _sc_gelu(x_hbm, o_hbm, inb, outb):
    wid = lax.axis_index("s") * 2 + lax.axis_index("c")
    base = wid * _RPW

    def chunk_body(i, _):
        r0 = base + i * _CHW
        pltpu.sync_copy(x_hbm.at[pl.ds(r0, _CHW), :], inb)

        def sub_body(j, _):
            x = inb[pl.ds(j * _SUB, _SUB), :]
            e = jnp.exp(x * (_S1 + _S2 * (x * x)))
            outb[pl.ds(j * _SUB, _SUB), :] = x / (1.0 + e)
            return 0

        lax.fori_loop(0, _CHW // _SUB, sub_body, 0)
        pltpu.sync_copy(outb, o_hbm.at[pl.ds(r0, _CHW), :])
        return 0

    lax.fori_loop(0, _RPW // _CHW, chunk_body, 0)


def kernel(x, log_k_blend):
    B, T, D = x.shape
    x2 = x.reshape(B * T, D)
    out = _sc_gelu(x2)
    return out.reshape(B, T, D)


# speedup vs baseline: 2.0448x; 2.0448x over previous
import functools
import math

import jax
import jax.numpy as jnp
from jax import lax
from jax.experimental import pallas as pl
from jax.experimental.pallas import tpu as pltpu
from jax.experimental.pallas import tpu_sc as plsc

_C = math.sqrt(2.0 / math.pi)
_K = _C * 0.044715
_S1 = -2.0 * _C
_S2 = -2.0 * _C * 0.044715

_D = 2048
_ROWS = 8192
_TC_ROWS = 7168
_SC_ROWS = _ROWS - _TC_ROWS

_CH = 256
_NBUF = 4

_NW = 32
_RPW = _SC_ROWS // _NW
_CHW = 8
_SUB = 2


def _gelu(x):
    u = x * (_C + _K * (x * x))
    h = 0.5 * x
    return h + h * jnp.tanh(u)


def _tc_body(x_hbm, o_hbm, inb, outb, in_sem, out_sem):
    nchunks = _TC_ROWS // _CH
    rounds = nchunks // _NBUF

    def start_in(chunk, slot):
        pltpu.make_async_copy(
            x_hbm.at[pl.ds(chunk * _CH, _CH), :], inb.at[slot], in_sem.at[slot]
        ).start()

    def wait_in(chunk, slot):
        pltpu.make_async_copy(
            x_hbm.at[pl.ds(chunk * _CH, _CH), :], inb.at[slot], in_sem.at[slot]
        ).wait()

    def start_out(chunk, slot):
        pltpu.make_async_copy(
            outb.at[slot], o_hbm.at[pl.ds(chunk * _CH, _CH), :], out_sem.at[slot]
        ).start()

    def wait_out(chunk, slot):
        pltpu.make_async_copy(
            outb.at[slot], o_hbm.at[pl.ds(chunk * _CH, _CH), :], out_sem.at[slot]
        ).wait()

    for s in range(_NBUF):
        start_in(s, s)

    def round_body(r, _):
        for s in range(_NBUF):
            chunk = r * _NBUF + s
            wait_in(chunk, s)

            @pl.when(r > 0)
            def _():
                wait_out(chunk - _NBUF, s)

            outb[s] = _gelu(inb[s])
            start_out(chunk, s)

            @pl.when(r < rounds - 1)
            def _():
                start_in(chunk + _NBUF, s)

        return 0

    jax.lax.fori_loop(0, rounds, round_body, 0)

    for s in range(_NBUF):
        wait_out(nchunks - _NBUF + s, s)


def _tc_gelu(x2):
    return pl.pallas_call(
        _tc_body,
        in_specs=[pl.BlockSpec(memory_space=pltpu.HBM)],
        out_specs=pl.BlockSpec(memory_space=pltpu.HBM),
        out_shape=jax.ShapeDtypeStruct((_TC_ROWS, _D), jnp.float32),
        scratch_shapes=[
            pltpu.VMEM((_NBUF, _CH, _D), jnp.float32),
            pltpu.VMEM((_NBUF, _CH, _D), jnp.float32),
            pltpu.SemaphoreType.DMA((_NBUF,)),
            pltpu.SemaphoreType.DMA((_NBUF,)),
        ],
    )(x2)


@functools.partial(
    pl.kernel,
    mesh=plsc.VectorSubcoreMesh(core_axis_name="c", subcore_axis_name="s"),
    out_type=jax.ShapeDtypeStruct((_SC_ROWS, _D), jnp.float32),
    scratch_types=[
        pltpu.VMEM((_CHW, _D), jnp.float32),
        pltpu.VMEM((_CHW, _D), jnp.float32),
    ],
)
def _sc_gelu(x_hbm, o_hbm, inb, outb):
    wid = lax.axis_index("s") * 2 + lax.axis_index("c")
    base = _TC_ROWS + wid * _RPW

    def chunk_body(i, _):
        r0 = base + i * _CHW
        pltpu.sync_copy(x_hbm.at[pl.ds(r0, _CHW), :], inb)

        def sub_body(j, _):
            x = inb[pl.ds(j * _SUB, _SUB), :]
            e = jnp.exp(x * (_S1 + _S2 * (x * x)))
            outb[pl.ds(j * _SUB, _SUB), :] = x / (1.0 + e)
            return 0

        lax.fori_loop(0, _CHW // _SUB, sub_body, 0)
        pltpu.sync_copy(outb, o_hbm.at[pl.ds(r0, _CHW), :])
        return 0

    lax.fori_loop(0, _RPW // _CHW, chunk_body, 0)


def kernel(x, log_k_blend):
    B, T, D = x.shape
    x2 = x.reshape(B * T, D)
    y_tc = _tc_gelu(x2)
    y_sc = _sc_gelu(x2)
    out = jnp.concatenate([y_tc, y_sc], axis=0)
    return out.reshape(B, T, D)


# speedup vs baseline: 4.6134x; 2.2562x over previous
import math

import jax
import jax.numpy as jnp
from jax.experimental import pallas as pl
from jax.experimental.pallas import tpu as pltpu

_C = math.sqrt(2.0 / math.pi)
_K = _C * 0.044715

_D = 2048
_CH = 512
_NBUF = 2


def _gelu(x):
    u = x * (_C + _K * (x * x))
    h = 0.5 * x
    return h + h * jnp.tanh(u)


def _pipeline_body(x_hbm, o_hbm, inb, outb, in_sem, out_sem):
    n_rows = x_hbm.shape[0]
    nchunks = n_rows // _CH
    rounds = nchunks // _NBUF

    def start_in(chunk, slot):
        pltpu.make_async_copy(
            x_hbm.at[pl.ds(chunk * _CH, _CH), :], inb.at[slot], in_sem.at[slot]
        ).start()

    def wait_in(chunk, slot):
        pltpu.make_async_copy(
            x_hbm.at[pl.ds(chunk * _CH, _CH), :], inb.at[slot], in_sem.at[slot]
        ).wait()

    def start_out(chunk, slot):
        pltpu.make_async_copy(
            outb.at[slot], o_hbm.at[pl.ds(chunk * _CH, _CH), :], out_sem.at[slot]
        ).start()

    def wait_out(chunk, slot):
        pltpu.make_async_copy(
            outb.at[slot], o_hbm.at[pl.ds(chunk * _CH, _CH), :], out_sem.at[slot]
        ).wait()

    for s in range(_NBUF):
        start_in(s, s)

    def round_body(r, _):
        for s in range(_NBUF):
            chunk = r * _NBUF + s
            wait_in(chunk, s)

            @pl.when(r > 0)
            def _():
                wait_out(chunk - _NBUF, s)

            outb[s] = _gelu(inb[s])
            start_out(chunk, s)

            @pl.when(r < rounds - 1)
            def _():
                start_in(chunk + _NBUF, s)

        return 0

    jax.lax.fori_loop(0, rounds, round_body, 0)

    for s in range(_NBUF):
        wait_out(nchunks - _NBUF + s, s)


def kernel(x, log_k_blend):
    B, T, D = x.shape
    R = B * T
    x2 = x.reshape(R, D)
    out = pl.pallas_call(
        _pipeline_body,
        in_specs=[pl.BlockSpec(memory_space=pltpu.HBM)],
        out_specs=pl.BlockSpec(memory_space=pltpu.HBM),
        out_shape=jax.ShapeDtypeStruct((R, D), x.dtype),
        scratch_shapes=[
            pltpu.VMEM((_NBUF, _CH, _D), jnp.float32),
            pltpu.VMEM((_NBUF, _CH, _D), jnp.float32),
            pltpu.SemaphoreType.DMA((_NBUF,)),
            pltpu.SemaphoreType.DMA((_NBUF,)),
        ],
    )(x2)
    return out.reshape(B, T, D)


# speedup vs baseline: 4.9453x; 1.0720x over previous
import math

import jax
import jax.numpy as jnp
from jax.experimental import pallas as pl
from jax.experimental.pallas import tpu as pltpu

_C = math.sqrt(2.0 / math.pi)
_K = _C * 0.044715

_D = 2048
_CH = 256
_NBUF = 8


def _gelu(x):
    u = x * (_C + _K * (x * x))
    h = 0.5 * x
    return h + h * jnp.tanh(u)


def _pipeline_body(x_hbm, o_hbm, inb, outb, in_sem, out_sem):
    n_rows = x_hbm.shape[0]
    nchunks = n_rows // _CH
    rounds = nchunks // _NBUF

    def start_in(chunk, slot):
        pltpu.make_async_copy(
            x_hbm.at[pl.ds(chunk * _CH, _CH), :], inb.at[slot], in_sem.at[slot]
        ).start()

    def wait_in(chunk, slot):
        pltpu.make_async_copy(
            x_hbm.at[pl.ds(chunk * _CH, _CH), :], inb.at[slot], in_sem.at[slot]
        ).wait()

    def start_out(chunk, slot):
        pltpu.make_async_copy(
            outb.at[slot], o_hbm.at[pl.ds(chunk * _CH, _CH), :], out_sem.at[slot]
        ).start()

    def wait_out(chunk, slot):
        pltpu.make_async_copy(
            outb.at[slot], o_hbm.at[pl.ds(chunk * _CH, _CH), :], out_sem.at[slot]
        ).wait()

    for s in range(_NBUF):
        start_in(s, s)

    def round_body(r, _):
        for s in range(_NBUF):
            chunk = r * _NBUF + s
            wait_in(chunk, s)

            @pl.when(r > 0)
            def _():
                wait_out(chunk - _NBUF, s)

            outb[s] = _gelu(inb[s])
            start_out(chunk, s)

            @pl.when(r < rounds - 1)
            def _():
                start_in(chunk + _NBUF, s)

        return 0

    jax.lax.fori_loop(0, rounds, round_body, 0)

    for s in range(_NBUF):
        wait_out(nchunks - _NBUF + s, s)


def kernel(x, log_k_blend):
    B, T, D = x.shape
    R = B * T
    x2 = x.reshape(R, D)
    out = pl.pallas_call(
        _pipeline_body,
        in_specs=[pl.BlockSpec(memory_space=pltpu.HBM)],
        out_specs=pl.BlockSpec(memory_space=pltpu.HBM),
        out_shape=jax.ShapeDtypeStruct((R, D), x.dtype),
        scratch_shapes=[
            pltpu.VMEM((_NBUF, _CH, _D), jnp.float32),
            pltpu.VMEM((_NBUF, _CH, _D), jnp.float32),
            pltpu.SemaphoreType.DMA((_NBUF,)),
            pltpu.SemaphoreType.DMA((_NBUF,)),
        ],
    )(x2)
    return out.reshape(B, T, D)
